# baseline (speedup 1.0000x reference)
import functools

import jax
import jax.numpy as jnp
from jax.experimental import pallas as pl
from jax.experimental.pallas import tpu as pltpu


def _densify_body(fov0t_ref, w0t_ref, fov1_ref, w1_ref, a0_ref, a1t_ref):
    oh, taps = fov0t_ref.shape
    h = a0_ref.shape[1]
    w_in, ow = a1t_ref.shape
    col = jax.lax.broadcasted_iota(jnp.int32, (oh, h), 1)
    acc0 = jnp.zeros((oh, h), jnp.float32)
    for p in range(taps):
        acc0 += jnp.where(fov0t_ref[:, p : p + 1] == col,
                          w0t_ref[:, p : p + 1], 0.0)
    a0_ref[...] = acc0
    row = jax.lax.broadcasted_iota(jnp.int32, (w_in, ow), 0)
    acc1 = jnp.zeros((w_in, ow), jnp.float32)
    for p in range(taps):
        acc1 += jnp.where(fov1_ref[p : p + 1, :] == row,
                          w1_ref[p : p + 1, :], 0.0)
    a1t_ref[...] = acc1


def _resize_body(x0_ref, x1_ref, x2_ref, x3_ref, a0_ref, a1t_ref, o_ref):
    y = jnp.zeros((a0_ref.shape[0], x0_ref.shape[2]), jnp.float32)
    for q, x_ref in enumerate((x0_ref, x1_ref, x2_ref, x3_ref)):
        kq = x_ref.shape[1]
        y += jnp.dot(a0_ref[:, q * kq : (q + 1) * kq], x_ref[0],
                     preferred_element_type=jnp.float32)
    o_ref[0] = jnp.dot(y, a1t_ref[...], preferred_element_type=jnp.float32)


@functools.partial(jax.jit, static_argnames=("h", "w", "interpret"))
def _densify(fov0t, w0t, fov1, w1, h, w, interpret=False):
    oh = fov0t.shape[0]
    ow = fov1.shape[1]
    return pl.pallas_call(
        _densify_body,
        out_shape=(
            jax.ShapeDtypeStruct((oh, h), jnp.float32),
            jax.ShapeDtypeStruct((w, ow), jnp.float32),
        ),
        interpret=interpret,
    )(fov0t, w0t, fov1, w1)


@functools.partial(jax.jit, static_argnames=("interpret",))
def _resize(x, a0, a1t, interpret=False):
    n, h, w = x.shape
    oh = a0.shape[0]
    ow = a1t.shape[1]
    hq = h // 4
    quarter_specs = [
        pl.BlockSpec((1, hq, w), lambda i, q=q: (i, q, 0)) for q in range(4)
    ]
    return pl.pallas_call(
        _resize_body,
        grid=(n,),
        in_specs=quarter_specs + [
            pl.BlockSpec((oh, h), lambda i: (0, 0)),
            pl.BlockSpec((w, ow), lambda i: (0, 0)),
        ],
        out_specs=pl.BlockSpec((1, oh, ow), lambda i: (i, 0, 0)),
        out_shape=jax.ShapeDtypeStruct((n, oh, ow), jnp.float32),
        compiler_params=pltpu.CompilerParams(
            dimension_semantics=("parallel",),
        ),
        interpret=interpret,
    )(x, x, x, x, a0, a1t)


def kernel(in_tensor, w0, w1, fov0, fov1, interpret=False):
    b, c, h, w = in_tensor.shape
    taps, oh = fov0.shape
    ow = fov1.shape[1]
    fov0t = fov0.astype(jnp.int32).T
    w0t = w0.reshape(taps, oh).astype(jnp.float32).T
    a0, a1t = _densify(fov0t, w0t, fov1.astype(jnp.int32),
                       w1.reshape(taps, ow).astype(jnp.float32),
                       h, w, interpret=interpret)
    x = in_tensor.reshape(b * c, h, w)
    out = _resize(x, a0, a1t, interpret=interpret)
    return out.reshape(b, c, oh, ow)


# speedup vs baseline: 1.1707x; 1.1707x over previous
import functools

import jax
import jax.numpy as jnp
from jax.experimental import pallas as pl
from jax.experimental.pallas import tpu as pltpu


def _densify_body(fov0_ref, w0_ref, fov1_ref, w1_ref, a0t_ref, a1t_ref):
    taps = fov0_ref.shape[0]

    def build_t(fov_ref, w_ref, out_ref):
        in_len, out_len = out_ref.shape
        row = jax.lax.broadcasted_iota(jnp.int32, (in_len, out_len), 0)
        acc = jnp.zeros((in_len, out_len), jnp.float32)
        for p in range(taps):
            acc += jnp.where(fov_ref[p : p + 1, :] == row,
                             w_ref[p : p + 1, :], 0.0)
        out_ref[...] = acc

    build_t(fov0_ref, w0_ref, a0t_ref)
    build_t(fov1_ref, w1_ref, a1t_ref)


def _resize_body(x_ref, a0t_ref, a1t_ref, o_ref):
    nb = x_ref.shape[0]
    for j in range(nb):
        y = jax.lax.dot_general(
            a0t_ref[...], x_ref[j],
            dimension_numbers=(((0,), (0,)), ((), ())),
            preferred_element_type=jnp.float32,
        )
        o_ref[j] = jnp.dot(y, a1t_ref[...], preferred_element_type=jnp.float32)


@functools.partial(jax.jit, static_argnames=("h", "w", "interpret"))
def _densify(fov0, w0, fov1, w1, h, w, interpret=False):
    oh = fov0.shape[1]
    ow = fov1.shape[1]
    return pl.pallas_call(
        _densify_body,
        out_shape=(
            jax.ShapeDtypeStruct((h, oh), jnp.float32),
            jax.ShapeDtypeStruct((w, ow), jnp.float32),
        ),
        interpret=interpret,
    )(fov0, w0, fov1, w1)


@functools.partial(jax.jit, static_argnames=("block", "interpret"))
def _resize(x, a0t, a1t, block=1, interpret=False):
    n, h, w = x.shape
    oh = a0t.shape[1]
    ow = a1t.shape[1]
    return pl.pallas_call(
        _resize_body,
        grid=(n // block,),
        in_specs=[
            pl.BlockSpec((block, h, w), lambda i: (i, 0, 0)),
            pl.BlockSpec((h, oh), lambda i: (0, 0)),
            pl.BlockSpec((w, ow), lambda i: (0, 0)),
        ],
        out_specs=pl.BlockSpec((block, oh, ow), lambda i: (i, 0, 0)),
        out_shape=jax.ShapeDtypeStruct((n, oh, ow), jnp.float32),
        compiler_params=pltpu.CompilerParams(
            dimension_semantics=("parallel",),
        ),
        interpret=interpret,
    )(x, a0t, a1t)


def kernel(in_tensor, w0, w1, fov0, fov1, interpret=False):
    b, c, h, w = in_tensor.shape
    taps, oh = fov0.shape
    ow = fov1.shape[1]
    a0t, a1t = _densify(fov0.astype(jnp.int32),
                        w0.reshape(taps, oh).astype(jnp.float32),
                        fov1.astype(jnp.int32),
                        w1.reshape(taps, ow).astype(jnp.float32),
                        h, w, interpret=interpret)
    x = in_tensor.reshape(b * c, h, w)
    out = _resize(x, a0t, a1t, block=2, interpret=interpret)
    return out.reshape(b, c, oh, ow)


# speedup vs baseline: 1.2890x; 1.1010x over previous
import functools

import jax
import jax.numpy as jnp
from jax.experimental import pallas as pl
from jax.experimental.pallas import tpu as pltpu


def _resize_body(fov0_ref, w0_ref, fov1_ref, w1_ref, x_ref, o_ref,
                 a0t_ref, a1t_ref):
    taps = fov0_ref.shape[0]

    @pl.when(pl.program_id(0) == 0)
    def _densify():
        for fov_ref, w_ref, out_ref in ((fov0_ref, w0_ref, a0t_ref),
                                        (fov1_ref, w1_ref, a1t_ref)):
            in_len, out_len = out_ref.shape
            row = jax.lax.broadcasted_iota(jnp.int32, (in_len, out_len), 0)
            acc = jnp.zeros((in_len, out_len), jnp.float32)
            for p in range(taps):
                acc += jnp.where(fov_ref[p : p + 1, :] == row,
                                 w_ref[p : p + 1, :], 0.0)
            out_ref[...] = acc

    nb = x_ref.shape[0]
    for j in range(nb):
        y = jax.lax.dot_general(
            a0t_ref[...], x_ref[j],
            dimension_numbers=(((0,), (0,)), ((), ())),
            preferred_element_type=jnp.float32,
        )
        o_ref[j] = jnp.dot(y, a1t_ref[...], preferred_element_type=jnp.float32)


@functools.partial(jax.jit, static_argnames=("block", "interpret"))
def _resize(fov0, w0, fov1, w1, x, block=4, interpret=False):
    n, h, w = x.shape
    taps, oh = fov0.shape
    ow = fov1.shape[1]
    return pl.pallas_call(
        _resize_body,
        grid=(n // block,),
        in_specs=[
            pl.BlockSpec((taps, oh), lambda i: (0, 0)),
            pl.BlockSpec((taps, oh), lambda i: (0, 0)),
            pl.BlockSpec((taps, ow), lambda i: (0, 0)),
            pl.BlockSpec((taps, ow), lambda i: (0, 0)),
            pl.BlockSpec((block, h, w), lambda i: (i, 0, 0)),
        ],
        out_specs=pl.BlockSpec((block, oh, ow), lambda i: (i, 0, 0)),
        out_shape=jax.ShapeDtypeStruct((n, oh, ow), jnp.float32),
        scratch_shapes=[
            pltpu.VMEM((h, oh), jnp.float32),
            pltpu.VMEM((w, ow), jnp.float32),
        ],
        compiler_params=pltpu.CompilerParams(
            dimension_semantics=("arbitrary",),
        ),
        interpret=interpret,
    )(fov0, w0, fov1, w1, x)


def kernel(in_tensor, w0, w1, fov0, fov1, interpret=False):
    b, c, h, w = in_tensor.shape
    taps, oh = fov0.shape
    ow = fov1.shape[1]
    x = in_tensor.reshape(b * c, h, w)
    out = _resize(fov0.astype(jnp.int32),
                  w0.reshape(taps, oh).astype(jnp.float32),
                  fov1.astype(jnp.int32),
                  w1.reshape(taps, ow).astype(jnp.float32),
                  x, block=4, interpret=interpret)
    return out.reshape(b, c, oh, ow)
